# Initial kernel scaffold; baseline (speedup 1.0000x reference)
import jax
import jax.numpy as jnp
from jax.experimental import pallas as pl


def kernel(tgt, W):
    raise NotImplementedError("write your pallas kernel here")


# speedup vs baseline: 6.0756x; 6.0756x over previous
import numpy as np
import jax
import jax.numpy as jnp
from jax.experimental import pallas as pl

_B = 16
_T = 64
_K = 16
_P = 8
_F = _T * _P
_CH = 8
_NC = _T // _CH


def _axis_counts():
    c = np.zeros(_F, np.float64)
    x = np.arange(_T)[:, None]
    i = np.arange(_K)[None, :]
    g = np.clip(x * _P + i, 0, _F - 1)
    np.add.at(c, g.reshape(-1), 1.0)
    return (1.0 / c).astype(np.float32)

_INV_C = _axis_counts()


def _decode_body(x_ref, w_ref, icx_ref, icy_ref, o_ref):
    w = w_ref[...]
    icy = icy_ref[...]

    for c in range(_NC):
        r0 = c * _CH
        lo_row = max(r0 - 1, 0)
        nr = r0 + _CH - lo_row
        xs = x_ref[0, lo_row * _T:(r0 + _CH) * _T, :]
        t = jnp.dot(xs, w, preferred_element_type=jnp.float32)
        t4 = t.reshape(nr, _T, _K, _K)

        lo = t4[:, :, :, 0:_P]
        hi = t4[:, :, :, _P:]
        hi_sh = jnp.concatenate(
            [jnp.zeros((nr, 1, _K, _P), jnp.float32), hi[:, :-1]], axis=1)
        fy = lo + hi_sh
        ey = hi[:, _T - 1].sum(axis=-1)
        by_i = jax.lax.broadcasted_iota(jnp.int32, (nr, _T, _K, _P), 1)
        ry_i = jax.lax.broadcasted_iota(jnp.int32, (nr, _T, _K, _P), 3)
        my = jnp.logical_and(by_i == _T - 1, ry_i == _P - 1).astype(jnp.float32)
        fy = fy + ey[:, None, :, None] * my

        fy = fy.transpose(0, 2, 1, 3)
        if c == 0:
            fy = jnp.concatenate(
                [jnp.zeros((1, _K, _T, _P), jnp.float32), fy], axis=0)

        flo = fy[1:, 0:_P]
        fhi = fy[:-1, _P:]
        f = flo + fhi
        if c == _NC - 1:
            ex = fy[_CH, _P:].sum(axis=0)
            bx_i = jax.lax.broadcasted_iota(jnp.int32, (_CH, _P, _T, _P), 0)
            rx_i = jax.lax.broadcasted_iota(jnp.int32, (_CH, _P, _T, _P), 1)
            mx = jnp.logical_and(
                bx_i == _CH - 1, rx_i == _P - 1).astype(jnp.float32)
            f = f + ex[None, None] * mx

        rows = _CH * _P
        f = f.reshape(rows, _F)
        icx = icx_ref[c * rows:(c + 1) * rows, :]
        o_ref[0, c * rows:(c + 1) * rows, :] = f * icx * icy


def kernel(tgt, W):
    b = tgt.shape[0]
    w = W[0].reshape(_K * _K, -1).T
    icx = jnp.asarray(_INV_C).reshape(_F, 1)
    icy = jnp.asarray(_INV_C).reshape(1, _F)
    out = pl.pallas_call(
        _decode_body,
        grid=(b,),
        in_specs=[
            pl.BlockSpec((1, _T * _T, 128), lambda i: (i, 0, 0)),
            pl.BlockSpec((128, _K * _K), lambda i: (0, 0)),
            pl.BlockSpec((_F, 1), lambda i: (0, 0)),
            pl.BlockSpec((1, _F), lambda i: (0, 0)),
        ],
        out_specs=pl.BlockSpec((1, _F, _F), lambda i: (i, 0, 0)),
        out_shape=jax.ShapeDtypeStruct((b, _F, _F), jnp.float32),
    )(tgt, w, icx, icy)
    return out
